# baseline (device time: 108253 ns/iter reference)
import jax
import jax.numpy as jnp
from jax import lax
from jax.experimental import pallas as pl
from jax.experimental.pallas import tpu as pltpu

N_DEV = 8
STRIP = 512


def kernel(x, W1, W2):
    m, k = x.shape
    h_dim = W1.shape[1]
    n = W2.shape[1]
    ch = m // N_DEV
    nh = n // 2

    def body(x_ref, w1_ref, w2_ref, out_ref,
             xb, w1b, w2b, hb, hcache, stripA, stripB,
             copyA_sems, copyB_sems,
             txA, rxA, txB, rxB, txC, rxC, txD, rxD, txZ, rxZ,
             sendA, recvA, sendB, recvB,
             sendC, recvC, sendD, recvD, sendZ, recvZ,
             creditA, creditB, creditC, creditD, creditZ):
        my = lax.axis_index("i")
        left = lax.rem(my - 1 + N_DEV, N_DEV)
        right = lax.rem(my + 1, N_DEV)

        def cidx(off):
            return lax.rem(my + off + 4 * N_DEV, N_DEV)

        def chunk(idx):
            return pl.ds(idx * ch, ch)

        jobsA = [(w1_ref.at[:, pl.ds(j * STRIP, STRIP)], w1b, j * STRIP)
                 for j in range(h_dim // STRIP)]
        jobsA += [(x_ref.at[:, pl.ds(j * STRIP, STRIP)], xb, j * STRIP)
                  for j in range(k // STRIP)]
        jobsB = [(w2_ref.at[pl.ds(j * STRIP, STRIP), :], w2b, j * STRIP)
                 for j in range(h_dim // STRIP)]

        def strip_copy(jobs, buf, sems, j):
            return pltpu.make_async_copy(jobs[j][0], buf.at[j % 2],
                                         sems.at[j % 2])

        cpsA = [strip_copy(jobsA, stripA, copyA_sems, j) for j in range(2)]
        cpsB = [strip_copy(jobsB, stripB, copyB_sems, j) for j in range(2)]
        for cp in cpsA + cpsB:
            cp.start()

        barrier_sem = pltpu.get_barrier_semaphore()
        for nbr in (left, right, cidx(4)):
            pl.semaphore_signal(barrier_sem, inc=1, device_id=(nbr,),
                                device_id_type=pl.DeviceIdType.MESH)
        pl.semaphore_wait(barrier_sem, 3)

        for j in range(len(jobsA)):
            cpsA[j].wait()
            dst, off = jobsA[j][1], jobsA[j][2]
            dst[:, pl.ds(off, STRIP)] = stripA[j % 2].astype(jnp.bfloat16)
            if j + 2 < len(jobsA):
                cp = strip_copy(jobsA, stripA, copyA_sems, j + 2)
                cp.start()
                cpsA.append(cp)
        def h_of(c):
            return jnp.maximum(
                jnp.dot(xb[chunk(c), :], w1b[:, :],
                        preferred_element_type=jnp.float32), 0.0
            ).astype(jnp.bfloat16)

        hb[:, :] = h_of(my)
        for j in range(len(jobsB)):
            cpsB[j].wait()
            dst, off = jobsB[j][1], jobsB[j][2]
            dst[pl.ds(off, STRIP), :] = stripB[j % 2].astype(jnp.bfloat16)
            if j + 2 < len(jobsB):
                cp = strip_copy(jobsB, stripB, copyB_sems, j + 2)
                cp.start()
                cpsB.append(cp)
            if j == 1:
                hcache[0, :, :] = h_of(cidx(1))
            elif j == 3:
                hcache[3, :, :] = h_of(cidx(-1))

        def compute_chunk(c):
            rows = chunk(c)
            hc = jnp.dot(xb[rows, :], w1b[:, :],
                         preferred_element_type=jnp.float32)
            hc = jnp.maximum(hc, 0.0).astype(jnp.bfloat16)
            out_ref[rows, :] = jnp.dot(hc, w2b[:, :],
                                       preferred_element_type=jnp.float32)

        out_ref[chunk(my), :] = jnp.dot(hb[:, :], w2b[:, :],
                                        preferred_element_type=jnp.float32)

        def make_stream(tx, rx, ssems, rsems, credit, tgt, up, total):
            msgs = []
            ncons = [0]

            def send(src_chunk, col):
                j = len(msgs)
                if j >= 2:
                    msgs[j - 2].wait_send()
                    pl.semaphore_wait(credit, 1)
                tx[j % 2, :, :] = out_ref[chunk(src_chunk),
                                          pl.ds(col, nh)].astype(
                    jnp.bfloat16)
                r = pltpu.make_async_remote_copy(
                    src_ref=tx.at[j % 2], dst_ref=rx.at[j % 2],
                    send_sem=ssems.at[j % 2], recv_sem=rsems.at[j % 2],
                    device_id=(tgt,), device_id_type=pl.DeviceIdType.MESH)
                r.start()
                msgs.append(r)

            def consume(dst_chunk, col, accumulate):
                j = ncons[0]
                msgs[j].wait_recv()
                val = rx[j % 2, :, :].astype(jnp.float32)
                rows, cols = chunk(dst_chunk), pl.ds(col, nh)
                if accumulate:
                    out_ref[rows, cols] = out_ref[rows, cols] + val
                else:
                    out_ref[rows, cols] = val
                if j + 2 < total:
                    pl.semaphore_signal(credit, inc=1, device_id=(up,),
                                        device_id_type=pl.DeviceIdType.MESH)
                ncons[0] += 1

            def drain():
                msgs[-2].wait_send()
                msgs[-1].wait_send()

            return send, consume, drain

        LOW, HIGH = 0, nh
        sendA_, consA, drainA = make_stream(
            txA, rxA, sendA, recvA, creditA, right, left, 10)
        sendB_, consB, drainB = make_stream(
            txB, rxB, sendB, recvB, creditB, left, right, 10)
        sendC_, consC, drainC = make_stream(
            txC, rxC, sendC, recvC, creditC, left, right, 3)
        sendD_, consD, drainD = make_stream(
            txD, rxD, sendD, recvD, creditD, right, left, 3)
        sendZ_, consZ, drainZ = make_stream(
            txZ, rxZ, sendZ, recvZ, creditZ, cidx(4), cidx(4), 2)

        def half_dot(hval, dst_chunk, col):
            out_ref[chunk(dst_chunk), pl.ds(col, nh)] = jnp.dot(
                hval, w2b[:, pl.ds(col, nh)],
                preferred_element_type=jnp.float32)

        for s in range(N_DEV - 1):
            sendA_(cidx(-s), LOW)
            sendB_(cidx(s), HIGH)
            if s == 0:
                half_dot(hcache[0, :, :], cidx(1), HIGH)
                half_dot(hcache[3, :, :], cidx(-1), LOW)
            elif s < 3:
                hv = h_of(cidx(s + 1))
                hcache[s, :, :] = hv
                half_dot(hv, cidx(s + 1), HIGH)
                hv = h_of(cidx(-s - 1))
                hcache[3 + s, :, :] = hv
                half_dot(hv, cidx(-s - 1), LOW)
            elif s == 3:
                compute_chunk(cidx(4))
            else:
                q = 6 - s
                half_dot(hcache[q, :, :], cidx(q + 1), LOW)
                half_dot(hcache[3 + q, :, :], cidx(-q - 1), HIGH)
            consA(cidx(-s - 1), LOW, True)
            consB(cidx(s + 1), HIGH, True)


        sendZ_(cidx(1), LOW)
        sendZ_(cidx(-1), HIGH)
        for t in range(3):
            sendA_(cidx(1 - t), LOW)
            sendB_(cidx(t - 1), HIGH)
            sendC_(cidx(1 + t), LOW)
            sendD_(cidx(-1 - t), HIGH)
            consA(cidx(-t), LOW, False)
            consB(cidx(t), HIGH, False)
            consC(cidx(2 + t), LOW, False)
            consD(cidx(-2 - t), HIGH, False)
        consZ(cidx(5), LOW, False)
        consZ(cidx(3), HIGH, False)

        for drain in (drainA, drainB, drainC, drainD, drainZ):
            drain()

    return pl.pallas_call(
        body,
        out_shape=jax.ShapeDtypeStruct((m, n), jnp.float32),
        in_specs=[pl.BlockSpec(memory_space=pltpu.HBM)] * 3,
        out_specs=pl.BlockSpec(memory_space=pltpu.VMEM),
        scratch_shapes=[
            pltpu.VMEM((m, k), jnp.bfloat16),
            pltpu.VMEM((k, h_dim), jnp.bfloat16),
            pltpu.VMEM((h_dim, n), jnp.bfloat16),
            pltpu.VMEM((m // N_DEV, h_dim), jnp.bfloat16),
            pltpu.VMEM((6, m // N_DEV, h_dim), jnp.bfloat16),
            pltpu.VMEM((2, m, STRIP), jnp.float32),
            pltpu.VMEM((2, STRIP, n), jnp.float32),
            pltpu.SemaphoreType.DMA((2,)),
            pltpu.SemaphoreType.DMA((2,)),
            pltpu.VMEM((2, ch, nh), jnp.bfloat16),
            pltpu.VMEM((2, ch, nh), jnp.bfloat16),
            pltpu.VMEM((2, ch, nh), jnp.bfloat16),
            pltpu.VMEM((2, ch, nh), jnp.bfloat16),
            pltpu.VMEM((2, ch, nh), jnp.bfloat16),
            pltpu.VMEM((2, ch, nh), jnp.bfloat16),
            pltpu.VMEM((2, ch, nh), jnp.bfloat16),
            pltpu.VMEM((2, ch, nh), jnp.bfloat16),
            pltpu.VMEM((2, ch, nh), jnp.bfloat16),
            pltpu.VMEM((2, ch, nh), jnp.bfloat16),
            pltpu.SemaphoreType.DMA((2,)),
            pltpu.SemaphoreType.DMA((2,)),
            pltpu.SemaphoreType.DMA((2,)),
            pltpu.SemaphoreType.DMA((2,)),
            pltpu.SemaphoreType.DMA((2,)),
            pltpu.SemaphoreType.DMA((2,)),
            pltpu.SemaphoreType.DMA((2,)),
            pltpu.SemaphoreType.DMA((2,)),
            pltpu.SemaphoreType.DMA((2,)),
            pltpu.SemaphoreType.DMA((2,)),
            pltpu.SemaphoreType.REGULAR,
            pltpu.SemaphoreType.REGULAR,
            pltpu.SemaphoreType.REGULAR,
            pltpu.SemaphoreType.REGULAR,
            pltpu.SemaphoreType.REGULAR,
        ],
        compiler_params=pltpu.CompilerParams(
            collective_id=0, vmem_limit_bytes=100 * 1024 * 1024),
    )(x, W1, W2)


# device time: 104948 ns/iter; 1.0315x vs baseline; 1.0315x over previous
import jax
import jax.numpy as jnp
from jax import lax
from jax.experimental import pallas as pl
from jax.experimental.pallas import tpu as pltpu

N_DEV = 8
STRIP = 512


def kernel(x, W1, W2):
    m, k = x.shape
    h_dim = W1.shape[1]
    n = W2.shape[1]
    ch = m // N_DEV
    nh = n // 2

    def body(x_ref, w1_ref, w2_ref, out_ref,
             xb, w1b, w2b, hb, hcache, stripA, stripB,
             copyA_sems, copyB_sems,
             txA, rxA, txB, rxB, txC, rxC, txD, rxD, txZ, rxZ,
             sendA, recvA, sendB, recvB,
             sendC, recvC, sendD, recvD, sendZ, recvZ,
             creditA, creditB, creditC, creditD, creditZ):
        my = lax.axis_index("i")
        left = lax.rem(my - 1 + N_DEV, N_DEV)
        right = lax.rem(my + 1, N_DEV)

        def cidx(off):
            return lax.rem(my + off + 4 * N_DEV, N_DEV)

        def chunk(idx):
            return pl.ds(idx * ch, ch)

        jobsA = [(w1_ref.at[:, pl.ds(j * STRIP, STRIP)], w1b, j * STRIP)
                 for j in range(h_dim // STRIP)]
        jobsA += [(x_ref.at[:, pl.ds(j * STRIP, STRIP)], xb, j * STRIP)
                  for j in range(k // STRIP)]
        jobsB = [(w2_ref.at[pl.ds(j * STRIP, STRIP), :], w2b, j * STRIP)
                 for j in range(h_dim // STRIP)]

        def strip_copy(jobs, buf, sems, j):
            return pltpu.make_async_copy(jobs[j][0], buf.at[j % 2],
                                         sems.at[j % 2])

        cpsA = [strip_copy(jobsA, stripA, copyA_sems, j) for j in range(2)]
        cpsB = [strip_copy(jobsB, stripB, copyB_sems, j) for j in range(2)]
        for cp in cpsA + cpsB:
            cp.start()

        barrier_sem = pltpu.get_barrier_semaphore()
        for nbr in (left, right, cidx(4)):
            pl.semaphore_signal(barrier_sem, inc=1, device_id=(nbr,),
                                device_id_type=pl.DeviceIdType.MESH)
        pl.semaphore_wait(barrier_sem, 3)

        for j in range(len(jobsA)):
            cpsA[j].wait()
            dst, off = jobsA[j][1], jobsA[j][2]
            dst[:, pl.ds(off, STRIP)] = stripA[j % 2].astype(jnp.bfloat16)
            if j + 2 < len(jobsA):
                cp = strip_copy(jobsA, stripA, copyA_sems, j + 2)
                cp.start()
                cpsA.append(cp)
        def h_of(c):
            return jnp.maximum(
                jnp.dot(xb[chunk(c), :], w1b[:, :],
                        preferred_element_type=jnp.float32), 0.0
            ).astype(jnp.bfloat16)

        hb[:, :] = h_of(my)
        for j in range(len(jobsB)):
            cpsB[j].wait()
            dst, off = jobsB[j][1], jobsB[j][2]
            dst[pl.ds(off, STRIP), :] = stripB[j % 2].astype(jnp.bfloat16)
            if j + 2 < len(jobsB):
                cp = strip_copy(jobsB, stripB, copyB_sems, j + 2)
                cp.start()
                cpsB.append(cp)

        def compute_chunk(c):
            rows = chunk(c)
            hc = jnp.dot(xb[rows, :], w1b[:, :],
                         preferred_element_type=jnp.float32)
            hc = jnp.maximum(hc, 0.0).astype(jnp.bfloat16)
            out_ref[rows, :] = jnp.dot(hc, w2b[:, :],
                                       preferred_element_type=jnp.float32)

        out_ref[chunk(my), :] = jnp.dot(hb[:, :], w2b[:, :],
                                        preferred_element_type=jnp.float32)

        def make_stream(tx, rx, ssems, rsems, credit, tgt, up, total):
            msgs = []
            ncons = [0]

            def send(src_chunk, col):
                j = len(msgs)
                if j >= 2:
                    msgs[j - 2].wait_send()
                    pl.semaphore_wait(credit, 1)
                tx[j % 2, :, :] = out_ref[chunk(src_chunk),
                                          pl.ds(col, nh)].astype(
                    jnp.bfloat16)
                r = pltpu.make_async_remote_copy(
                    src_ref=tx.at[j % 2], dst_ref=rx.at[j % 2],
                    send_sem=ssems.at[j % 2], recv_sem=rsems.at[j % 2],
                    device_id=(tgt,), device_id_type=pl.DeviceIdType.MESH)
                r.start()
                msgs.append(r)

            def consume(dst_chunk, col, accumulate):
                j = ncons[0]
                msgs[j].wait_recv()
                val = rx[j % 2, :, :].astype(jnp.float32)
                rows, cols = chunk(dst_chunk), pl.ds(col, nh)
                if accumulate:
                    out_ref[rows, cols] = out_ref[rows, cols] + val
                else:
                    out_ref[rows, cols] = val
                if j + 2 < total:
                    pl.semaphore_signal(credit, inc=1, device_id=(up,),
                                        device_id_type=pl.DeviceIdType.MESH)
                ncons[0] += 1

            def drain():
                msgs[-2].wait_send()
                msgs[-1].wait_send()

            return send, consume, drain

        LOW, HIGH = 0, nh
        sendA_, consA, drainA = make_stream(
            txA, rxA, sendA, recvA, creditA, right, left, 10)
        sendB_, consB, drainB = make_stream(
            txB, rxB, sendB, recvB, creditB, left, right, 10)
        sendC_, consC, drainC = make_stream(
            txC, rxC, sendC, recvC, creditC, left, right, 3)
        sendD_, consD, drainD = make_stream(
            txD, rxD, sendD, recvD, creditD, right, left, 3)
        sendZ_, consZ, drainZ = make_stream(
            txZ, rxZ, sendZ, recvZ, creditZ, cidx(4), cidx(4), 2)

        def half_dot(hval, dst_chunk, col):
            out_ref[chunk(dst_chunk), pl.ds(col, nh)] = jnp.dot(
                hval, w2b[:, pl.ds(col, nh)],
                preferred_element_type=jnp.float32)

        for s in range(N_DEV - 1):
            sendA_(cidx(-s), LOW)
            sendB_(cidx(s), HIGH)
            if s < 3:
                hv = h_of(cidx(s + 1))
                hcache[s, :, :] = hv
                half_dot(hv, cidx(s + 1), HIGH)
                hv = h_of(cidx(-s - 1))
                hcache[3 + s, :, :] = hv
                half_dot(hv, cidx(-s - 1), LOW)
            elif s == 3:
                compute_chunk(cidx(4))
            else:
                q = 6 - s
                half_dot(hcache[q, :, :], cidx(q + 1), LOW)
                half_dot(hcache[3 + q, :, :], cidx(-q - 1), HIGH)
            consA(cidx(-s - 1), LOW, True)
            consB(cidx(s + 1), HIGH, True)


        sendZ_(cidx(1), LOW)
        sendZ_(cidx(-1), HIGH)
        for t in range(3):
            sendA_(cidx(1 - t), LOW)
            sendB_(cidx(t - 1), HIGH)
            sendC_(cidx(1 + t), LOW)
            sendD_(cidx(-1 - t), HIGH)
            consA(cidx(-t), LOW, False)
            consB(cidx(t), HIGH, False)
            consC(cidx(2 + t), LOW, False)
            consD(cidx(-2 - t), HIGH, False)
        consZ(cidx(5), LOW, False)
        consZ(cidx(3), HIGH, False)

        for drain in (drainA, drainB, drainC, drainD, drainZ):
            drain()

    return pl.pallas_call(
        body,
        out_shape=jax.ShapeDtypeStruct((m, n), jnp.float32),
        in_specs=[pl.BlockSpec(memory_space=pltpu.HBM)] * 3,
        out_specs=pl.BlockSpec(memory_space=pltpu.VMEM),
        scratch_shapes=[
            pltpu.VMEM((m, k), jnp.bfloat16),
            pltpu.VMEM((k, h_dim), jnp.bfloat16),
            pltpu.VMEM((h_dim, n), jnp.bfloat16),
            pltpu.VMEM((m // N_DEV, h_dim), jnp.bfloat16),
            pltpu.VMEM((6, m // N_DEV, h_dim), jnp.bfloat16),
            pltpu.VMEM((2, m, STRIP), jnp.float32),
            pltpu.VMEM((2, STRIP, n), jnp.float32),
            pltpu.SemaphoreType.DMA((2,)),
            pltpu.SemaphoreType.DMA((2,)),
            pltpu.VMEM((2, ch, nh), jnp.bfloat16),
            pltpu.VMEM((2, ch, nh), jnp.bfloat16),
            pltpu.VMEM((2, ch, nh), jnp.bfloat16),
            pltpu.VMEM((2, ch, nh), jnp.bfloat16),
            pltpu.VMEM((2, ch, nh), jnp.bfloat16),
            pltpu.VMEM((2, ch, nh), jnp.bfloat16),
            pltpu.VMEM((2, ch, nh), jnp.bfloat16),
            pltpu.VMEM((2, ch, nh), jnp.bfloat16),
            pltpu.VMEM((2, ch, nh), jnp.bfloat16),
            pltpu.VMEM((2, ch, nh), jnp.bfloat16),
            pltpu.SemaphoreType.DMA((2,)),
            pltpu.SemaphoreType.DMA((2,)),
            pltpu.SemaphoreType.DMA((2,)),
            pltpu.SemaphoreType.DMA((2,)),
            pltpu.SemaphoreType.DMA((2,)),
            pltpu.SemaphoreType.DMA((2,)),
            pltpu.SemaphoreType.DMA((2,)),
            pltpu.SemaphoreType.DMA((2,)),
            pltpu.SemaphoreType.DMA((2,)),
            pltpu.SemaphoreType.DMA((2,)),
            pltpu.SemaphoreType.REGULAR,
            pltpu.SemaphoreType.REGULAR,
            pltpu.SemaphoreType.REGULAR,
            pltpu.SemaphoreType.REGULAR,
            pltpu.SemaphoreType.REGULAR,
        ],
        compiler_params=pltpu.CompilerParams(
            collective_id=0, vmem_limit_bytes=100 * 1024 * 1024),
    )(x, W1, W2)
